# baseline (speedup 1.0000x reference)
import functools

import jax
import jax.numpy as jnp
from jax import lax
from jax.experimental import pallas as pl
from jax.experimental.pallas import tpu as pltpu
from jax.experimental.pallas import tpu_sc as plsc

N = 10000
D = 128
DA = 144
E = 320000
NC = 2
NS = 16
CHUNK = 128
NBUF = 2
ROWS = 2512
ROWS_ALLOC = 2514
ROWS_PER_TILE = ROWS // NS
PAIRS_PER_TILE = ROWS_PER_TILE // 2
N_PAD = 10112
STRIPE = N_PAD // NS

_sc_mesh = plsc.VectorSubcoreMesh(core_axis_name="c", subcore_axis_name="s",
                                  num_cores=NC, num_subcores=NS)


@functools.partial(
    pl.kernel,
    out_type=[jax.ShapeDtypeStruct((N_PAD, DA), jnp.float32),
              jax.ShapeDtypeStruct((N_PAD, DA), jnp.float32)],
    mesh=_sc_mesh,
    scratch_types=[
        [pltpu.VMEM((2, CHUNK), jnp.int32) for _ in range(2)],
        [pltpu.VMEM((2, CHUNK), jnp.int32) for _ in range(2)],
        [pltpu.VMEM((CHUNK, DA), jnp.float32) for _ in range(NBUF)],
        pltpu.VMEM_SHARED((N_PAD, DA), jnp.float32),
        [pltpu.SemaphoreType.DMA for _ in range(NBUF)],
        [pltpu.SemaphoreType.DMA for _ in range(NBUF)],
        [pltpu.SemaphoreType.DMA for _ in range(2)],
    ],
    compiler_params=pltpu.CompilerParams(use_tc_tiling_on_sc=False),
)
def _sc_accumulate(xu_aug, xi_aug, src_ui, dst_ui, src_iu, dst_iu, zeros,
                   acc_item, acc_user, sidx, didx, rows, acc_sh, gsems,
                   ssems, isems):
    c = lax.axis_index("c")
    s = lax.axis_index("s")

    pltpu.sync_copy(zeros.at[pl.ds(s * STRIPE, STRIPE)],
                    acc_sh.at[pl.ds(s * STRIPE, STRIPE)])
    plsc.subcore_barrier()

    def run_edges(src_hbm, dst_hbm, x_hbm):
        base = s * ROWS_PER_TILE

        def start_idx(pair_row, p):
            pltpu.async_copy(src_hbm.at[pl.ds(pair_row, 2)], sidx[p],
                             isems[p])
            pltpu.async_copy(dst_hbm.at[pl.ds(pair_row, 2)], didx[p],
                             isems[p])

        def drain_idx(p):
            pltpu.make_async_copy(src_hbm.at[pl.ds(0, 2)], sidx[p],
                                  isems[p]).wait()
            pltpu.make_async_copy(src_hbm.at[pl.ds(0, 2)], didx[p],
                                  isems[p]).wait()

        def drain_scatter(k):
            pltpu.make_async_copy(x_hbm.at[pl.ds(0, CHUNK)], rows[k],
                                  ssems[k]).wait()

        def do_pair(i, p, first):
            drain_idx(p)
            if first is None:
                drain_scatter(0)
            else:
                @pl.when(first > 0)
                def _():
                    drain_scatter(0)
            ga = pltpu.async_copy(x_hbm.at[sidx[p].at[0]], rows[0], gsems[0])
            if first is None:
                drain_scatter(1)
            else:
                @pl.when(first > 0)
                def _():
                    drain_scatter(1)
            start_idx(base + (i + 1) * 2, 1 - p)
            gb = pltpu.async_copy(x_hbm.at[sidx[p].at[1]], rows[1], gsems[1])
            ga.wait()
            pltpu.async_copy(rows[0], acc_sh.at[didx[p].at[0]], ssems[0],
                             add=True)
            gb.wait()
            pltpu.async_copy(rows[1], acc_sh.at[didx[p].at[1]], ssems[1],
                             add=True)

        start_idx(base, 0)

        def body(i, carry):
            do_pair(2 * i, 0, first=i)
            do_pair(2 * i + 1, 1, first=None)
            return carry
        lax.fori_loop(0, PAIRS_PER_TILE // 2, body, 0)
        drain_idx(0)
        drain_scatter(0)
        gd = pltpu.async_copy(x_hbm.at[sidx[0].at[0]], rows[0], gsems[0])
        drain_scatter(1)
        gd.wait()
        pltpu.sync_copy(rows[0], acc_sh.at[didx[0].at[0]], add=True)

    @pl.when(c == 0)
    def _():
        run_edges(src_ui, dst_ui, xu_aug)

    @pl.when(c == 1)
    def _():
        run_edges(src_iu, dst_iu, xi_aug)

    plsc.subcore_barrier()

    @pl.when(c == 0)
    def _():
        pltpu.sync_copy(acc_sh.at[pl.ds(s * STRIPE, STRIPE)],
                        acc_item.at[pl.ds(s * STRIPE, STRIPE)])

    @pl.when(c == 1)
    def _():
        pltpu.sync_copy(acc_sh.at[pl.ds(s * STRIPE, STRIPE)],
                        acc_user.at[pl.ds(s * STRIPE, STRIPE)])


def _tc_body(acc_u, acc_i, xu, xi, wl_iu, wr_iu, b_iu, wl_ui, wr_ui, b_ui,
             out):
    for t, (acc, xd, wl, wr, b) in enumerate((
            (acc_u, xu, wl_iu, wr_iu, b_iu),
            (acc_i, xi, wl_ui, wr_ui, b_ui))):
        summed = acc[:N, :D]
        cnt = acc[:N, D:D + 1]
        mean = summed / jnp.maximum(cnt, 1.0)
        out[t] = (jnp.dot(mean, wl[...], preferred_element_type=jnp.float32)
                  + jnp.dot(xd[...], wr[...], preferred_element_type=jnp.float32)
                  + b[...])


def kernel(x_user, x_item, edge_index_rates, edge_index_rev,
           W_l_ui, W_r_ui, b_ui, W_l_iu, W_r_iu, b_iu):
    ones16 = jnp.ones((N, DA - D), jnp.float32)
    xu_aug = jnp.concatenate([x_user, ones16], axis=1)
    xi_aug = jnp.concatenate([x_item, ones16], axis=1)

    pad = ROWS_ALLOC * CHUNK - E

    def pad_edges(ei):
        src = jnp.concatenate([ei[0].astype(jnp.int32),
                               jnp.zeros((pad,), jnp.int32)])
        dst = jnp.concatenate([ei[1].astype(jnp.int32),
                               jnp.full((pad,), N, jnp.int32)])
        return src.reshape(ROWS_ALLOC, CHUNK), dst.reshape(ROWS_ALLOC, CHUNK)

    src_ui, dst_ui = pad_edges(edge_index_rates)
    src_iu, dst_iu = pad_edges(edge_index_rev)
    zeros = jnp.zeros((N_PAD, DA), jnp.float32)

    acc_item, acc_user = _sc_accumulate(xu_aug, xi_aug, src_ui, dst_ui,
                                        src_iu, dst_iu, zeros)

    out = pl.pallas_call(
        _tc_body,
        out_shape=jax.ShapeDtypeStruct((2, N, D), jnp.float32),
    )(acc_user, acc_item, x_user, x_item,
      W_l_iu, W_r_iu, b_iu.reshape(1, D),
      W_l_ui, W_r_ui, b_ui.reshape(1, D))
    return out


# speedup vs baseline: 1.2391x; 1.2391x over previous
import functools

import jax
import jax.numpy as jnp
from jax import lax
from jax.experimental import pallas as pl
from jax.experimental.pallas import tpu as pltpu
from jax.experimental.pallas import tpu_sc as plsc

N = 10000
D = 128
DA = 128
E = 320000
NC = 2
NS = 16
CHUNK = 128
NBUF = 2
ROWS = 2512
ROWS_ALLOC = 2514
ROWS_PER_TILE = ROWS // NS
PAIRS_PER_TILE = ROWS_PER_TILE // 2
N_PAD = 10112
STRIPE = N_PAD // NS

_sc_mesh = plsc.VectorSubcoreMesh(core_axis_name="c", subcore_axis_name="s",
                                  num_cores=NC, num_subcores=NS)


@functools.partial(
    pl.kernel,
    out_type=[jax.ShapeDtypeStruct((N_PAD, DA), jnp.float32),
              jax.ShapeDtypeStruct((N_PAD, DA), jnp.float32),
              jax.ShapeDtypeStruct((NS, N_PAD), jnp.float32),
              jax.ShapeDtypeStruct((NS, N_PAD), jnp.float32)],
    mesh=_sc_mesh,
    scratch_types=[
        [pltpu.VMEM((2, CHUNK), jnp.int32) for _ in range(2)],
        [pltpu.VMEM((2, CHUNK), jnp.int32) for _ in range(2)],
        [pltpu.VMEM((CHUNK, DA), jnp.float32) for _ in range(NBUF)],
        pltpu.VMEM((N_PAD,), jnp.float32),
        pltpu.VMEM_SHARED((N_PAD, DA), jnp.float32),
        [pltpu.SemaphoreType.DMA for _ in range(NBUF)],
        [pltpu.SemaphoreType.DMA for _ in range(NBUF)],
        [pltpu.SemaphoreType.DMA for _ in range(2)],
    ],
    compiler_params=pltpu.CompilerParams(use_tc_tiling_on_sc=False,
                                         needs_layout_passes=False),
)
def _sc_accumulate(xu, xi, src_ui, dst_ui, src_iu, dst_iu, zeros, zeros1d,
                   acc_item, acc_user, cnt_item, cnt_user, sidx, didx, rows,
                   hist, acc_sh, gsems, ssems, isems):
    c = lax.axis_index("c")
    s = lax.axis_index("s")

    pltpu.sync_copy(zeros.at[pl.ds(s * STRIPE, STRIPE)],
                    acc_sh.at[pl.ds(s * STRIPE, STRIPE)])
    pltpu.sync_copy(zeros1d, hist)
    plsc.subcore_barrier()

    def run_edges(src_hbm, dst_hbm, x_hbm):
        base = s * ROWS_PER_TILE

        def start_idx(pair_row, p):
            pltpu.async_copy(src_hbm.at[pl.ds(pair_row, 2)], sidx[p],
                             isems[p])
            pltpu.async_copy(dst_hbm.at[pl.ds(pair_row, 2)], didx[p],
                             isems[p])

        def drain_idx(p):
            pltpu.make_async_copy(src_hbm.at[pl.ds(0, 2)], sidx[p],
                                  isems[p]).wait()
            pltpu.make_async_copy(src_hbm.at[pl.ds(0, 2)], didx[p],
                                  isems[p]).wait()

        def drain_scatter(k):
            pltpu.make_async_copy(x_hbm.at[pl.ds(0, CHUNK)], rows[k],
                                  ssems[k]).wait()

        ones_v = jnp.full((16,), 1.0, jnp.float32)

        def hist_update(p, k):
            for v in range(CHUNK // 16):
                idxv = didx[p][k, pl.ds(v * 16, 16)]
                plsc.addupdate_scatter(hist, [idxv], ones_v)

        def do_pair(i, p, first):
            drain_idx(p)
            if first is None:
                drain_scatter(0)
            else:
                @pl.when(first > 0)
                def _():
                    drain_scatter(0)
            ga = pltpu.async_copy(x_hbm.at[sidx[p].at[0]], rows[0], gsems[0])
            if first is None:
                drain_scatter(1)
            else:
                @pl.when(first > 0)
                def _():
                    drain_scatter(1)
            start_idx(base + (i + 1) * 2, 1 - p)
            gb = pltpu.async_copy(x_hbm.at[sidx[p].at[1]], rows[1], gsems[1])
            hist_update(p, 0)
            hist_update(p, 1)
            ga.wait()
            pltpu.async_copy(rows[0], acc_sh.at[didx[p].at[0]], ssems[0],
                             add=True)
            gb.wait()
            pltpu.async_copy(rows[1], acc_sh.at[didx[p].at[1]], ssems[1],
                             add=True)

        start_idx(base, 0)

        def body(i, carry):
            do_pair(2 * i, 0, first=i)
            do_pair(2 * i + 1, 1, first=None)
            return carry
        lax.fori_loop(0, PAIRS_PER_TILE // 2, body, 0)
        drain_idx(0)
        drain_scatter(0)
        gd = pltpu.async_copy(x_hbm.at[sidx[0].at[0]], rows[0], gsems[0])
        hist_update(0, 0)
        drain_scatter(1)
        gd.wait()
        pltpu.sync_copy(rows[0], acc_sh.at[didx[0].at[0]], add=True)

    @pl.when(c == 0)
    def _():
        run_edges(src_ui, dst_ui, xu)

    @pl.when(c == 1)
    def _():
        run_edges(src_iu, dst_iu, xi)

    plsc.subcore_barrier()

    @pl.when(c == 0)
    def _():
        pltpu.sync_copy(acc_sh.at[pl.ds(s * STRIPE, STRIPE)],
                        acc_item.at[pl.ds(s * STRIPE, STRIPE)])
        pltpu.sync_copy(hist, cnt_item.at[s])

    @pl.when(c == 1)
    def _():
        pltpu.sync_copy(acc_sh.at[pl.ds(s * STRIPE, STRIPE)],
                        acc_user.at[pl.ds(s * STRIPE, STRIPE)])
        pltpu.sync_copy(hist, cnt_user.at[s])


def _tc_body(acc_u, acc_i, cnt_u, cnt_i, xu, xi,
             wl_iu, wr_iu, b_iu, wl_ui, wr_ui, b_ui, out):
    for t, (acc, cnt_t, xd, wl, wr, b) in enumerate((
            (acc_u, cnt_u, xu, wl_iu, wr_iu, b_iu),
            (acc_i, cnt_i, xi, wl_ui, wr_ui, b_ui))):
        summed = acc[:N, :]
        cnt = jnp.sum(cnt_t[...], axis=0)[:N, None]
        mean = summed / jnp.maximum(cnt, 1.0)
        out[t] = (jnp.dot(mean, wl[...], preferred_element_type=jnp.float32)
                  + jnp.dot(xd[...], wr[...], preferred_element_type=jnp.float32)
                  + b[...])


def kernel(x_user, x_item, edge_index_rates, edge_index_rev,
           W_l_ui, W_r_ui, b_ui, W_l_iu, W_r_iu, b_iu):
    pad = ROWS_ALLOC * CHUNK - E

    def pad_edges(ei):
        src = jnp.concatenate([ei[0].astype(jnp.int32),
                               jnp.zeros((pad,), jnp.int32)])
        dst = jnp.concatenate([ei[1].astype(jnp.int32),
                               jnp.full((pad,), N, jnp.int32)])
        return src.reshape(ROWS_ALLOC, CHUNK), dst.reshape(ROWS_ALLOC, CHUNK)

    src_ui, dst_ui = pad_edges(edge_index_rates)
    src_iu, dst_iu = pad_edges(edge_index_rev)
    zeros = jnp.zeros((N_PAD, DA), jnp.float32)
    zeros1d = jnp.zeros((N_PAD,), jnp.float32)

    acc_item, acc_user, cnt_item, cnt_user = _sc_accumulate(
        x_user, x_item, src_ui, dst_ui, src_iu, dst_iu, zeros, zeros1d)

    out = pl.pallas_call(
        _tc_body,
        out_shape=jax.ShapeDtypeStruct((2, N, D), jnp.float32),
    )(acc_user, acc_item, cnt_user, cnt_item, x_user, x_item,
      W_l_iu, W_r_iu, b_iu.reshape(1, D),
      W_l_ui, W_r_ui, b_ui.reshape(1, D))
    return out


# speedup vs baseline: 1.2493x; 1.0083x over previous
import functools

import jax
import jax.numpy as jnp
from jax import lax
from jax.experimental import pallas as pl
from jax.experimental.pallas import tpu as pltpu
from jax.experimental.pallas import tpu_sc as plsc

N = 10000
D = 128
DA = 128
E = 320000
NC = 2
NS = 16
CHUNK = 96
NBUF = 3
ROWS = 3360
ROWS_ALLOC = 3363
ROWS_PER_TILE = ROWS // NS
GRPS_PER_TILE = ROWS_PER_TILE // NBUF
N_PAD = 10112
STRIPE = N_PAD // NS

_sc_mesh = plsc.VectorSubcoreMesh(core_axis_name="c", subcore_axis_name="s",
                                  num_cores=NC, num_subcores=NS)


@functools.partial(
    pl.kernel,
    out_type=[jax.ShapeDtypeStruct((N_PAD, DA), jnp.float32),
              jax.ShapeDtypeStruct((N_PAD, DA), jnp.float32),
              jax.ShapeDtypeStruct((NS, N_PAD), jnp.float32),
              jax.ShapeDtypeStruct((NS, N_PAD), jnp.float32)],
    mesh=_sc_mesh,
    scratch_types=[
        [pltpu.VMEM((NBUF, CHUNK), jnp.int32) for _ in range(2)],
        [pltpu.VMEM((NBUF, CHUNK), jnp.int32) for _ in range(2)],
        [pltpu.VMEM((CHUNK, DA), jnp.float32) for _ in range(NBUF)],
        pltpu.VMEM((N_PAD,), jnp.float32),
        pltpu.VMEM_SHARED((N_PAD, DA), jnp.float32),
        [pltpu.SemaphoreType.DMA for _ in range(NBUF)],
        [pltpu.SemaphoreType.DMA for _ in range(NBUF)],
        [pltpu.SemaphoreType.DMA for _ in range(2)],
    ],
    compiler_params=pltpu.CompilerParams(use_tc_tiling_on_sc=False,
                                         needs_layout_passes=False),
)
def _sc_accumulate(xu, xi, src_ui, dst_ui, src_iu, dst_iu, zeros, zeros1d,
                   acc_item, acc_user, cnt_item, cnt_user, sidx, didx, rows,
                   hist, acc_sh, gsems, ssems, isems):
    c = lax.axis_index("c")
    s = lax.axis_index("s")

    pltpu.sync_copy(zeros.at[pl.ds(s * STRIPE, STRIPE)],
                    acc_sh.at[pl.ds(s * STRIPE, STRIPE)])
    pltpu.sync_copy(zeros1d, hist)
    plsc.subcore_barrier()

    def run_edges(src_hbm, dst_hbm, x_hbm):
        base = s * ROWS_PER_TILE

        def start_idx(grp_row, p):
            pltpu.async_copy(src_hbm.at[pl.ds(grp_row, NBUF)], sidx[p],
                             isems[p])
            pltpu.async_copy(dst_hbm.at[pl.ds(grp_row, NBUF)], didx[p],
                             isems[p])

        def drain_idx(p):
            pltpu.make_async_copy(src_hbm.at[pl.ds(0, NBUF)], sidx[p],
                                  isems[p]).wait()
            pltpu.make_async_copy(src_hbm.at[pl.ds(0, NBUF)], didx[p],
                                  isems[p]).wait()

        def drain_scatter(k):
            pltpu.make_async_copy(x_hbm.at[pl.ds(0, CHUNK)], rows[k],
                                  ssems[k]).wait()

        ones_v = jnp.full((16,), 1.0, jnp.float32)

        def hist_update(p, k):
            for v in range(CHUNK // 16):
                idxv = didx[p][k, pl.ds(v * 16, 16)]
                plsc.addupdate_scatter(hist, [idxv], ones_v)

        def do_group(i, p, first):
            drain_idx(p)
            gds = []
            for k in range(NBUF):
                if first is None:
                    drain_scatter(k)
                else:
                    @pl.when(first > 0)
                    def _(k=k):
                        drain_scatter(k)
                if k == NBUF - 1:
                    start_idx(base + (i + 1) * NBUF, 1 - p)
                gds.append(pltpu.async_copy(x_hbm.at[sidx[p].at[k]],
                                            rows[k], gsems[k]))
            for k in range(NBUF):
                hist_update(p, k)
            for k in range(NBUF):
                gds[k].wait()
                pltpu.async_copy(rows[k], acc_sh.at[didx[p].at[k]],
                                 ssems[k], add=True)

        start_idx(base, 0)

        def body(i, carry):
            do_group(2 * i, 0, first=i)
            do_group(2 * i + 1, 1, first=None)
            return carry
        lax.fori_loop(0, GRPS_PER_TILE // 2, body, 0)
        drain_idx(0)
        for k in range(NBUF):
            drain_scatter(k)

    @pl.when(c == 0)
    def _():
        run_edges(src_ui, dst_ui, xu)

    @pl.when(c == 1)
    def _():
        run_edges(src_iu, dst_iu, xi)

    plsc.subcore_barrier()

    @pl.when(c == 0)
    def _():
        pltpu.sync_copy(acc_sh.at[pl.ds(s * STRIPE, STRIPE)],
                        acc_item.at[pl.ds(s * STRIPE, STRIPE)])
        pltpu.sync_copy(hist, cnt_item.at[s])

    @pl.when(c == 1)
    def _():
        pltpu.sync_copy(acc_sh.at[pl.ds(s * STRIPE, STRIPE)],
                        acc_user.at[pl.ds(s * STRIPE, STRIPE)])
        pltpu.sync_copy(hist, cnt_user.at[s])


def _tc_body(acc_u, acc_i, cnt_u, cnt_i, xu, xi,
             wl_iu, wr_iu, b_iu, wl_ui, wr_ui, b_ui, out):
    for t, (acc, cnt_t, xd, wl, wr, b) in enumerate((
            (acc_u, cnt_u, xu, wl_iu, wr_iu, b_iu),
            (acc_i, cnt_i, xi, wl_ui, wr_ui, b_ui))):
        summed = acc[:N, :]
        cnt = jnp.sum(cnt_t[...], axis=0)[:N, None]
        mean = summed / jnp.maximum(cnt, 1.0)
        out[t] = (jnp.dot(mean, wl[...], preferred_element_type=jnp.float32)
                  + jnp.dot(xd[...], wr[...], preferred_element_type=jnp.float32)
                  + b[...])


def kernel(x_user, x_item, edge_index_rates, edge_index_rev,
           W_l_ui, W_r_ui, b_ui, W_l_iu, W_r_iu, b_iu):
    pad = ROWS_ALLOC * CHUNK - E

    def pad_edges(ei):
        src = jnp.concatenate([ei[0].astype(jnp.int32),
                               jnp.zeros((pad,), jnp.int32)])
        dst = jnp.concatenate([ei[1].astype(jnp.int32),
                               jnp.full((pad,), N, jnp.int32)])
        return src.reshape(ROWS_ALLOC, CHUNK), dst.reshape(ROWS_ALLOC, CHUNK)

    src_ui, dst_ui = pad_edges(edge_index_rates)
    src_iu, dst_iu = pad_edges(edge_index_rev)
    zeros = jnp.zeros((N_PAD, DA), jnp.float32)
    zeros1d = jnp.zeros((N_PAD,), jnp.float32)

    acc_item, acc_user, cnt_item, cnt_user = _sc_accumulate(
        x_user, x_item, src_ui, dst_ui, src_iu, dst_iu, zeros, zeros1d)

    out = pl.pallas_call(
        _tc_body,
        out_shape=jax.ShapeDtypeStruct((2, N, D), jnp.float32),
    )(acc_user, acc_item, cnt_user, cnt_item, x_user, x_item,
      W_l_iu, W_r_iu, b_iu.reshape(1, D),
      W_l_ui, W_r_ui, b_ui.reshape(1, D))
    return out
